# baseline (speedup 1.0000x reference)
import functools

import jax
import jax.numpy as jnp
from jax import lax
from jax.experimental import pallas as pl
from jax.experimental.pallas import tpu as pltpu
from jax.experimental.pallas import tpu_sc as plsc

B, P, H, W, C = 4, 8192, 128, 128, 130
STEPS, FDIM, HID = 3, 132, 64
CPAD = 144
NW = 32
CHUNK = (B * P) // NW
SUB = 64
NSUB = CHUNK // SUB


def _bilinear_prep(x, y, hs, ws, base):
    xs = x / hs * H
    ys = y / ws * W
    x0 = jnp.floor(xs)
    x1 = x0 + 1.0
    y0 = jnp.floor(ys)
    y1 = y0 + 1.0
    w00 = (x1 - xs) * (y1 - ys)
    w01 = (x1 - xs) * (ys - y0)
    w10 = (xs - x0) * (y1 - ys)
    w11 = (xs - x0) * (ys - y0)
    x0c = jnp.clip(x0, 0.0, H - 1.0)
    x1c = jnp.clip(x1, 0.0, H - 1.0)
    y0c = jnp.clip(y0, 0.0, W - 1.0)
    y1c = jnp.clip(y1, 0.0, W - 1.0)

    def flat(xi, yi):
        return (xi * W + yi).astype(jnp.int32) + base

    idx4 = jnp.stack([flat(x0c, y0c), flat(x0c, y1c),
                      flat(x1c, y0c), flat(x1c, y1c)], axis=0)
    w4 = jnp.stack([w00, w01, w10, w11], axis=0)
    return idx4, w4


def _prep_body(coords_ref, bbox_ref, idx_ref, w_ref):
    b = pl.program_id(0)
    x = coords_ref[0, :, 0]
    y = coords_ref[0, :, 1]
    idx4, w4 = _bilinear_prep(x, y, bbox_ref[b, 3], bbox_ref[b, 2],
                              b * (H * W))
    idx_ref[0] = idx4
    w_ref[0] = w4


def _ring_mean(y):
    n = y.shape[0]
    s = None
    for d in (-4, -3, -2, -1, 1, 2, 3, 4):
        part = pltpu.roll(y, (-d) % n, 0)
        s = part if s is None else s + part
    return s * 0.125


def _gcn1_body(cnn_ref, w0_ref, b0_ref, h1_ref):
    x = cnn_ref[0]
    z = jnp.dot(x, w0_ref[...], preferred_element_type=jnp.float32)
    a = z[:, :HID] + b0_ref[0:1, :]
    h1_ref[0] = jax.nn.relu(a + _ring_mean(z[:, HID:]))


def _gcn2_body(h1_ref, hull_ref, bbox_ref, w1_ref, b1_ref, wo_ref, bo_ref,
               pred_ref, idx_ref, w_ref):
    b = pl.program_id(0)
    z2 = jnp.dot(h1_ref[0], w1_ref[...], preferred_element_type=jnp.float32)
    a2 = z2[:, :HID] + b1_ref[0:1, :]
    h2 = jax.nn.relu(a2 + _ring_mean(z2[:, HID:]))
    shift = jnp.dot(h2, wo_ref[...], preferred_element_type=jnp.float32)
    pred_x = hull_ref[0, 0, :] + shift[:, 0] + bo_ref[0, 0]
    pred_y = hull_ref[0, 1, :] + shift[:, 1] + bo_ref[0, 1]
    pred_ref[0, 0, :] = pred_x
    pred_ref[0, 1, :] = pred_y

    hs = bbox_ref[b, 3]
    ws = bbox_ref[b, 2]
    cx = pred_x * hs
    cy = pred_y * ws
    idx4, w4 = _bilinear_prep(cx, cy, hs, ws, b * (H * W))
    idx_ref[0] = idx4
    w_ref[0] = w4


def _interp_body(conv_hbm, idx_hbm, w_hbm, hull_hbm, out_hbm,
                 idx_v, w_v, hull_v, r0, r1, r2, r3, out_v, sem, sem_s):
    cid = lax.axis_index("c")
    sid = lax.axis_index("s")
    wid = cid * 16 + sid
    bw = wid // (P // CHUNK)
    poff = (wid % (P // CHUNK)) * CHUNK
    pltpu.sync_copy(idx_hbm.at[bw, :, pl.ds(poff, CHUNK)], idx_v)
    pltpu.sync_copy(w_hbm.at[bw, :, pl.ds(poff, CHUNK)], w_v)
    pltpu.sync_copy(hull_hbm.at[bw, :, pl.ds(poff, CHUNK)], hull_v)
    lane = lax.iota(jnp.int32, 16)
    zero16 = jnp.zeros((16,), jnp.float32)
    rbufs = (r0, r1, r2, r3)

    def fire(scn, par):
        for k in range(4):
            pltpu.async_copy(
                conv_hbm.at[idx_v.at[k, pl.ds(scn * SUB, SUB)]],
                rbufs[k].at[par], sem)

    fire(0, 0)

    def sub_body(scn, carry):
        par = lax.rem(scn, 2)
        for k in range(4):
            pltpu.make_async_copy(
                conv_hbm.at[idx_v.at[k, pl.ds(scn * SUB, SUB)]],
                rbufs[k].at[par], sem).wait()

        @pl.when(scn + 1 < NSUB)
        def _():
            fire(scn + 1, 1 - par)

        @pl.when(scn >= 2)
        def _():
            pltpu.make_async_copy(
                out_v.at[par], out_hbm.at[pl.ds(wid * CHUNK, SUB)],
                sem_s).wait()

        s0 = scn * SUB

        def grp_body(g, carry2):
            gbase = g * 16
            wv0 = w_v[0, pl.ds(s0 + gbase, 16)]
            wv1 = w_v[1, pl.ds(s0 + gbase, 16)]
            wv2 = w_v[2, pl.ds(s0 + gbase, 16)]
            wv3 = w_v[3, pl.ds(s0 + gbase, 16)]
            hxv = hull_v[0, pl.ds(s0 + gbase, 16)]
            hyv = hull_v[1, pl.ds(s0 + gbase, 16)]
            for j in range(16):
                p = gbase + j
                wb0 = jnp.full((16,), wv0[j])
                wb1 = jnp.full((16,), wv1[j])
                wb2 = jnp.full((16,), wv2[j])
                wb3 = jnp.full((16,), wv3[j])
                for cc in range(CPAD // 16):
                    sl = pl.ds(cc * 16, 16)
                    acc = r0[par, p, sl] * wb0 + r1[par, p, sl] * wb1
                    acc = acc + r2[par, p, sl] * wb2 + r3[par, p, sl] * wb3
                    if cc == (C // 16):
                        hb = jnp.where(lane == (C - 16 * cc),
                                       jnp.full((16,), hxv[j]), zero16)
                        hb = jnp.where(lane == (C + 1 - 16 * cc),
                                       jnp.full((16,), hyv[j]), hb)
                        acc = acc + hb
                    out_v[par, p, sl] = acc
            return carry2

        lax.fori_loop(0, SUB // 16, grp_body, 0)
        pltpu.async_copy(out_v.at[par],
                         out_hbm.at[pl.ds(wid * CHUNK + s0, SUB)], sem_s)
        return carry

    lax.fori_loop(0, NSUB, sub_body, 0)
    for _ in range(2):
        pltpu.make_async_copy(
            out_v.at[0], out_hbm.at[pl.ds(wid * CHUNK, SUB)], sem_s).wait()


def _make_interp():
    mesh = plsc.VectorSubcoreMesh(core_axis_name="c", subcore_axis_name="s")
    return functools.partial(
        pl.kernel,
        out_type=jax.ShapeDtypeStruct((B * P, CPAD), jnp.float32),
        mesh=mesh,
        scratch_types=[
            pltpu.VMEM((4, CHUNK), jnp.int32),
            pltpu.VMEM((4, CHUNK), jnp.float32),
            pltpu.VMEM((2, CHUNK), jnp.float32),
            pltpu.VMEM((2, SUB, CPAD), jnp.float32),
            pltpu.VMEM((2, SUB, CPAD), jnp.float32),
            pltpu.VMEM((2, SUB, CPAD), jnp.float32),
            pltpu.VMEM((2, SUB, CPAD), jnp.float32),
            pltpu.VMEM((2, SUB, CPAD), jnp.float32),
            pltpu.SemaphoreType.DMA,
            pltpu.SemaphoreType.DMA,
        ],
        compiler_params=pltpu.CompilerParams(use_tc_tiling_on_sc=False),
    )(_interp_body)


def _make_prep():
    return pl.pallas_call(
        _prep_body,
        grid=(B,),
        in_specs=[
            pl.BlockSpec((1, P, 2), lambda b: (b, 0, 0)),
            pl.BlockSpec(memory_space=pltpu.SMEM),
        ],
        out_specs=[
            pl.BlockSpec((1, 4, P), lambda b: (b, 0, 0)),
            pl.BlockSpec((1, 4, P), lambda b: (b, 0, 0)),
        ],
        out_shape=[
            jax.ShapeDtypeStruct((B, 4, P), jnp.int32),
            jax.ShapeDtypeStruct((B, 4, P), jnp.float32),
        ],
    )


def _make_gcn1():
    full = lambda b: (0, 0)
    return pl.pallas_call(
        _gcn1_body,
        grid=(B,),
        in_specs=[
            pl.BlockSpec((1, P, CPAD), lambda b: (b, 0, 0)),
            pl.BlockSpec((CPAD, 2 * HID), full),
            pl.BlockSpec((1, HID), full),
        ],
        out_specs=pl.BlockSpec((1, P, HID), lambda b: (b, 0, 0)),
        out_shape=jax.ShapeDtypeStruct((B, P, HID), jnp.float32),
    )


def _make_gcn2():
    full = lambda b: (0, 0)
    return pl.pallas_call(
        _gcn2_body,
        grid=(B,),
        in_specs=[
            pl.BlockSpec((1, P, HID), lambda b: (b, 0, 0)),
            pl.BlockSpec((1, 2, P), lambda b: (b, 0, 0)),
            pl.BlockSpec(memory_space=pltpu.SMEM),
            pl.BlockSpec((HID, 2 * HID), full),
            pl.BlockSpec((1, HID), full),
            pl.BlockSpec((HID, 2), full),
            pl.BlockSpec(memory_space=pltpu.SMEM),
        ],
        out_specs=[
            pl.BlockSpec((1, 2, P), lambda b: (b, 0, 0)),
            pl.BlockSpec((1, 4, P), lambda b: (b, 0, 0)),
            pl.BlockSpec((1, 4, P), lambda b: (b, 0, 0)),
        ],
        out_shape=[
            jax.ShapeDtypeStruct((B, 2, P), jnp.float32),
            jax.ShapeDtypeStruct((B, 4, P), jnp.int32),
            jax.ShapeDtypeStruct((B, 4, P), jnp.float32),
        ],
    )


def kernel(tg2, feature_hull, original_hull, binary_hull, bbox, dp,
           Ws0, Wn0, b0, Ws1, Wn1, b1, Wo, bo):
    del feature_hull, dp
    conv = jnp.pad(tg2, ((0, 0), (0, CPAD - C), (0, 0), (0, 0)))
    conv = jnp.transpose(conv, (0, 2, 3, 1)).reshape(B * H * W, CPAD)

    w0cat = jnp.concatenate([Ws0, Wn0], axis=2)
    w0full = jnp.pad(w0cat, ((0, 0), (0, CPAD - FDIM), (0, 0)))
    w1cat = jnp.concatenate([Ws1, Wn1], axis=2)

    prep = _make_prep()
    interp = _make_interp()
    gcn1 = _make_gcn1()
    gcn2 = _make_gcn2()

    idx4, w4 = prep(original_hull, bbox)
    hull_t = jnp.transpose(binary_hull, (0, 2, 1))
    preds = []
    for i in range(STEPS):
        cnn = interp(conv, idx4, w4, hull_t).reshape(B, P, CPAD)
        h1 = gcn1(cnn, w0full[i], b0[i][None, :])
        pred_t, idx4, w4 = gcn2(h1, hull_t, bbox,
                                w1cat[i], b1[i][None, :], Wo[i], bo[i][None, :])
        preds.append(jnp.transpose(pred_t, (0, 2, 1)))
        hull_t = pred_t
    return jnp.stack(preds)


# speedup vs baseline: 1.0535x; 1.0535x over previous
import functools

import jax
import jax.numpy as jnp
from jax import lax
from jax.experimental import pallas as pl
from jax.experimental.pallas import tpu as pltpu
from jax.experimental.pallas import tpu_sc as plsc

B, P, H, W, C = 4, 8192, 128, 128, 130
STEPS, FDIM, HID = 3, 132, 64
TW = 160
NW = 32
CHUNK = (B * P) // NW
SUB = 128
NSUB = CHUNK // SUB


def _bilinear_prep(x, y, hs, ws, base):
    xs = x / hs * H
    ys = y / ws * W
    x0 = jnp.floor(xs)
    x1 = x0 + 1.0
    y0 = jnp.floor(ys)
    y1 = y0 + 1.0
    w00 = (x1 - xs) * (y1 - ys)
    w01 = (x1 - xs) * (ys - y0)
    w10 = (xs - x0) * (y1 - ys)
    w11 = (xs - x0) * (ys - y0)
    x0c = jnp.clip(x0, 0.0, H - 1.0)
    x1c = jnp.clip(x1, 0.0, H - 1.0)
    y0c = jnp.clip(y0, 0.0, W - 1.0)
    y1c = jnp.clip(y1, 0.0, W - 1.0)

    def flat(xi, yi):
        return (xi * W + yi).astype(jnp.int32) + base

    idx4 = jnp.stack([flat(x0c, y0c), flat(x0c, y1c),
                      flat(x1c, y0c), flat(x1c, y1c)], axis=0)
    w4 = jnp.stack([w00, w01, w10, w11], axis=0)
    return idx4, w4


def _prep_body(coords_ref, bbox_ref, idx_ref, w_ref):
    b = pl.program_id(0)
    x = coords_ref[0, :, 0]
    y = coords_ref[0, :, 1]
    idx4, w4 = _bilinear_prep(x, y, bbox_ref[b, 3], bbox_ref[b, 2],
                              b * (H * W))
    idx_ref[0] = idx4
    w_ref[0] = w4


def _ring_mean(y):
    n = y.shape[0]
    s = None
    for d in (-4, -3, -2, -1, 1, 2, 3, 4):
        part = pltpu.roll(y, (-d) % n, 0)
        s = part if s is None else s + part
    return s * 0.125


def _gcn1_body(cnn_ref, w0_ref, b0_ref, h1_ref):
    x = cnn_ref[0]
    z = jnp.dot(x, w0_ref[...], preferred_element_type=jnp.float32)
    a = z[:, :HID] + b0_ref[0:1, :]
    h1_ref[0] = jax.nn.relu(a + _ring_mean(z[:, HID:]))


def _gcn2_body(h1_ref, hull_ref, bbox_ref, w1_ref, b1_ref, wo_ref, bo_ref,
               pred_ref, idx_ref, w_ref):
    b = pl.program_id(0)
    z2 = jnp.dot(h1_ref[0], w1_ref[...], preferred_element_type=jnp.float32)
    a2 = z2[:, :HID] + b1_ref[0:1, :]
    h2 = jax.nn.relu(a2 + _ring_mean(z2[:, HID:]))
    shift = jnp.dot(h2, wo_ref[...], preferred_element_type=jnp.float32)
    pred_x = hull_ref[0, 0, :] + shift[:, 0] + bo_ref[0, 0]
    pred_y = hull_ref[0, 1, :] + shift[:, 1] + bo_ref[0, 1]
    pred_ref[0, 0, :] = pred_x
    pred_ref[0, 1, :] = pred_y

    hs = bbox_ref[b, 3]
    ws = bbox_ref[b, 2]
    cx = pred_x * hs
    cy = pred_y * ws
    idx4, w4 = _bilinear_prep(cx, cy, hs, ws, b * (H * W))
    idx_ref[0] = idx4
    w_ref[0] = w4


def _interp_body(conv_hbm, idx_hbm, w_hbm, hull_hbm, hmask_hbm, out_hbm,
                 idx_v, w_v, hull_v, hm_v,
                 ra0, ra1, ra2, ra3, rb0, rb1, rb2, rb3, oa, ob,
                 sem, sem_sa, sem_sb):
    cid = lax.axis_index("c")
    sid = lax.axis_index("s")
    wid = cid * 16 + sid
    bw = wid // (P // CHUNK)
    poff = (wid % (P // CHUNK)) * CHUNK
    pltpu.sync_copy(idx_hbm.at[bw, :, pl.ds(poff, CHUNK)], idx_v)
    pltpu.sync_copy(w_hbm.at[bw, :, pl.ds(poff, CHUNK)], w_v)
    pltpu.sync_copy(hull_hbm.at[bw, :, pl.ds(poff, CHUNK)], hull_v)
    pltpu.sync_copy(hmask_hbm, hm_v)
    mx = hm_v[0, :]
    my = hm_v[1, :]
    bufs = {0: (ra0, ra1, ra2, ra3, oa, sem_sa),
            1: (rb0, rb1, rb2, rb3, ob, sem_sb)}

    def fire(scn, side):
        for k in range(4):
            pltpu.async_copy(
                conv_hbm.at[idx_v.at[k, pl.ds(scn * SUB, SUB)]],
                bufs[side][k], sem)

    def wait_gathers(scn, side):
        for k in range(4):
            pltpu.make_async_copy(
                conv_hbm.at[idx_v.at[k, pl.ds(scn * SUB, SUB)]],
                bufs[side][k], sem).wait()

    def compute(scn, side):
        r0, r1, r2, r3, ov, _ = bufs[side]
        s0 = scn * SUB

        def bcast(vec, j):
            return plsc.bitcast(jnp.full((16,), vec[j]), jnp.bfloat16)

        def grp_body(g, carry2):
            gbase = g * 16
            wv0 = w_v[0, pl.ds(s0 + gbase, 16)]
            wv1 = w_v[1, pl.ds(s0 + gbase, 16)]
            wv2 = w_v[2, pl.ds(s0 + gbase, 16)]
            wv3 = w_v[3, pl.ds(s0 + gbase, 16)]
            hxv = hull_v[0, pl.ds(s0 + gbase, 16)]
            hyv = hull_v[1, pl.ds(s0 + gbase, 16)]
            for j in range(16):
                p = gbase + j
                wb0 = bcast(wv0, j)
                wb1 = bcast(wv1, j)
                wb2 = bcast(wv2, j)
                wb3 = bcast(wv3, j)
                for cc in range(TW // 32):
                    sl = pl.ds(cc * 32, 32)
                    acc = r0[p, sl] * wb0 + r1[p, sl] * wb1
                    acc = acc + r2[p, sl] * wb2 + r3[p, sl] * wb3
                    if cc == 4:
                        acc = acc + mx * bcast(hxv, j)
                        acc = acc + my * bcast(hyv, j)
                    ov[p, sl] = acc
            return carry2

        lax.fori_loop(0, SUB // 16, grp_body, 0)

    def store(scn, side):
        ov, sem_s = bufs[side][4], bufs[side][5]
        pltpu.async_copy(ov, out_hbm.at[pl.ds(wid * CHUNK + scn * SUB, SUB)],
                         sem_s)

    def wait_store(side):
        ov, sem_s = bufs[side][4], bufs[side][5]
        pltpu.make_async_copy(ov, out_hbm.at[pl.ds(wid * CHUNK, SUB)],
                              sem_s).wait()

    fire(0, 0)
    fire(1, 1)

    def pair_body(gg, carry):
        g0 = 2 * gg
        for side in (0, 1):
            g = g0 + side
            wait_gathers(g, side)

            @pl.when(gg >= 1)
            def _():
                wait_store(side)

            compute(g, side)

            @pl.when(g + 2 < NSUB)
            def _():
                fire(g + 2, side)

            store(g, side)
        return carry

    lax.fori_loop(0, NSUB // 2, pair_body, 0)
    wait_store(0)
    wait_store(1)


def _make_interp():
    mesh = plsc.VectorSubcoreMesh(core_axis_name="c", subcore_axis_name="s")
    rbuf = pltpu.VMEM((SUB, TW), jnp.bfloat16)
    return functools.partial(
        pl.kernel,
        out_type=jax.ShapeDtypeStruct((B * P, TW), jnp.bfloat16),
        mesh=mesh,
        scratch_types=[
            pltpu.VMEM((4, CHUNK), jnp.int32),
            pltpu.VMEM((4, CHUNK), jnp.int32),
            pltpu.VMEM((2, CHUNK), jnp.int32),
            pltpu.VMEM((2, 32), jnp.bfloat16),
            rbuf, rbuf, rbuf, rbuf, rbuf, rbuf, rbuf, rbuf,
            rbuf, rbuf,
            pltpu.SemaphoreType.DMA,
            pltpu.SemaphoreType.DMA,
            pltpu.SemaphoreType.DMA,
        ],
        compiler_params=pltpu.CompilerParams(use_tc_tiling_on_sc=False,
                                             needs_layout_passes=False),
    )(_interp_body)


def _make_prep():
    return pl.pallas_call(
        _prep_body,
        grid=(B,),
        in_specs=[
            pl.BlockSpec((1, P, 2), lambda b: (b, 0, 0)),
            pl.BlockSpec(memory_space=pltpu.SMEM),
        ],
        out_specs=[
            pl.BlockSpec((1, 4, P), lambda b: (b, 0, 0)),
            pl.BlockSpec((1, 4, P), lambda b: (b, 0, 0)),
        ],
        out_shape=[
            jax.ShapeDtypeStruct((B, 4, P), jnp.int32),
            jax.ShapeDtypeStruct((B, 4, P), jnp.float32),
        ],
    )


def _make_gcn1():
    full = lambda b: (0, 0)
    return pl.pallas_call(
        _gcn1_body,
        grid=(B,),
        in_specs=[
            pl.BlockSpec((1, P, TW), lambda b: (b, 0, 0)),
            pl.BlockSpec((TW, 2 * HID), full),
            pl.BlockSpec((1, HID), full),
        ],
        out_specs=pl.BlockSpec((1, P, HID), lambda b: (b, 0, 0)),
        out_shape=jax.ShapeDtypeStruct((B, P, HID), jnp.float32),
    )


def _make_gcn2():
    full = lambda b: (0, 0)
    return pl.pallas_call(
        _gcn2_body,
        grid=(B,),
        in_specs=[
            pl.BlockSpec((1, P, HID), lambda b: (b, 0, 0)),
            pl.BlockSpec((1, 2, P), lambda b: (b, 0, 0)),
            pl.BlockSpec(memory_space=pltpu.SMEM),
            pl.BlockSpec((HID, 2 * HID), full),
            pl.BlockSpec((1, HID), full),
            pl.BlockSpec((HID, 2), full),
            pl.BlockSpec(memory_space=pltpu.SMEM),
        ],
        out_specs=[
            pl.BlockSpec((1, 2, P), lambda b: (b, 0, 0)),
            pl.BlockSpec((1, 4, P), lambda b: (b, 0, 0)),
            pl.BlockSpec((1, 4, P), lambda b: (b, 0, 0)),
        ],
        out_shape=[
            jax.ShapeDtypeStruct((B, 2, P), jnp.float32),
            jax.ShapeDtypeStruct((B, 4, P), jnp.int32),
            jax.ShapeDtypeStruct((B, 4, P), jnp.float32),
        ],
    )


def kernel(tg2, feature_hull, original_hull, binary_hull, bbox, dp,
           Ws0, Wn0, b0, Ws1, Wn1, b1, Wo, bo):
    del feature_hull, dp
    conv = jnp.pad(tg2.astype(jnp.bfloat16), ((0, 0), (0, TW - C), (0, 0), (0, 0)))
    conv = jnp.transpose(conv, (0, 2, 3, 1)).reshape(B * H * W, TW)

    w0cat = jnp.concatenate([Ws0, Wn0], axis=2)
    w0full = jnp.pad(w0cat, ((0, 0), (0, TW - FDIM), (0, 0))).astype(jnp.bfloat16)
    w1cat = jnp.concatenate([Ws1, Wn1], axis=2)

    prep = _make_prep()
    interp = _make_interp()
    gcn1 = _make_gcn1()
    gcn2 = _make_gcn2()

    hmask = jnp.zeros((2, 32), jnp.float32)
    hmask = hmask.at[0, C - 128].set(1.0).at[1, C + 1 - 128].set(1.0)
    hmask = hmask.astype(jnp.bfloat16)

    def dup16(x):
        u = jax.lax.bitcast_convert_type(
            x.astype(jnp.bfloat16), jnp.uint16).astype(jnp.uint32)
        return jax.lax.bitcast_convert_type(u * jnp.uint32(65537), jnp.int32)

    idx4, w4 = prep(original_hull, bbox)
    hull_t = jnp.transpose(binary_hull, (0, 2, 1))
    preds = []
    for i in range(STEPS):
        cnn = interp(conv, idx4, dup16(w4), dup16(hull_t),
                     hmask).reshape(B, P, TW)
        h1 = gcn1(cnn, w0full[i], b0[i][None, :])
        pred_t, idx4, w4 = gcn2(h1, hull_t, bbox,
                                w1cat[i], b1[i][None, :], Wo[i], bo[i][None, :])
        preds.append(jnp.transpose(pred_t, (0, 2, 1)))
        hull_t = pred_t
    return jnp.stack(preds)


# speedup vs baseline: 1.0958x; 1.0401x over previous
import functools

import jax
import jax.numpy as jnp
from jax import lax
from jax.experimental import pallas as pl
from jax.experimental.pallas import tpu as pltpu
from jax.experimental.pallas import tpu_sc as plsc

B, P, H, W, C = 4, 8192, 128, 128, 130
STEPS, FDIM, HID = 3, 132, 64
TW = 160
NW = 32
CHUNK = (B * P) // NW
SUB = 128
NSUB = CHUNK // SUB


def _bilinear_prep(x, y, hs, ws, base):
    xs = x / hs * H
    ys = y / ws * W
    x0 = jnp.floor(xs)
    x1 = x0 + 1.0
    y0 = jnp.floor(ys)
    y1 = y0 + 1.0
    w00 = (x1 - xs) * (y1 - ys)
    w01 = (x1 - xs) * (ys - y0)
    w10 = (xs - x0) * (y1 - ys)
    w11 = (xs - x0) * (ys - y0)
    x0c = jnp.clip(x0, 0.0, H - 1.0)
    x1c = jnp.clip(x1, 0.0, H - 1.0)
    y0c = jnp.clip(y0, 0.0, W - 1.0)
    y1c = jnp.clip(y1, 0.0, W - 1.0)

    def flat(xi, yi):
        return (xi * W + yi).astype(jnp.int32) + base

    idx4 = jnp.stack([flat(x0c, y0c), flat(x0c, y1c),
                      flat(x1c, y0c), flat(x1c, y1c)], axis=0)
    w4 = jnp.stack([w00, w01, w10, w11], axis=0)
    return idx4, w4


def _dup16(x):
    u = jax.lax.bitcast_convert_type(
        x.astype(jnp.bfloat16), jnp.uint16).astype(jnp.uint32)
    return jax.lax.bitcast_convert_type(u * jnp.uint32(65537), jnp.int32)


def _prep_body(coords_ref, bbox_ref, idx_ref, w_ref):
    b = pl.program_id(0)
    x = coords_ref[0, :, 0]
    y = coords_ref[0, :, 1]
    idx4, w4 = _bilinear_prep(x, y, bbox_ref[b, 3], bbox_ref[b, 2],
                              b * (H * W))
    idx_ref[0] = idx4
    w_ref[0] = _dup16(w4)


def _ring_mean(y):
    n = y.shape[0]
    s = None
    for d in (-4, -3, -2, -1, 1, 2, 3, 4):
        part = pltpu.roll(y, (-d) % n, 0)
        s = part if s is None else s + part
    return s * 0.125


def _gcn_body(cnn_ref, hull_ref, bbox_ref, w0_ref, b0_ref,
              w1_ref, b1_ref, wo_ref, bo_ref,
              pred_ref, idx_ref, w_ref, hld_ref):
    b = pl.program_id(0)
    x = cnn_ref[0]
    z = jnp.dot(x, w0_ref[...], preferred_element_type=jnp.float32)
    a = z[:, :HID] + b0_ref[0:1, :]
    h1 = jax.nn.relu(a + _ring_mean(z[:, HID:]))
    z2 = jnp.dot(h1, w1_ref[...], preferred_element_type=jnp.float32)
    a2 = z2[:, :HID] + b1_ref[0:1, :]
    h2 = jax.nn.relu(a2 + _ring_mean(z2[:, HID:]))
    shift = jnp.dot(h2, wo_ref[...], preferred_element_type=jnp.float32)
    pred_x = hull_ref[0, 0, :] + shift[:, 0] + bo_ref[0, 0]
    pred_y = hull_ref[0, 1, :] + shift[:, 1] + bo_ref[0, 1]
    pred_ref[0, 0, :] = pred_x
    pred_ref[0, 1, :] = pred_y
    hld_ref[0] = _dup16(jnp.stack([pred_x, pred_y], axis=0))

    hs = bbox_ref[b, 3]
    ws = bbox_ref[b, 2]
    cx = pred_x * hs
    cy = pred_y * ws
    idx4, w4 = _bilinear_prep(cx, cy, hs, ws, b * (H * W))
    idx_ref[0] = idx4
    w_ref[0] = _dup16(w4)


def _interp_body(conv_hbm, idx_hbm, w_hbm, hull_hbm, hmask_hbm, out_hbm,
                 idx_v, w_v, hull_v, hm_v,
                 ra0, ra1, ra2, ra3, rb0, rb1, rb2, rb3, oa, ob,
                 sem, sem_sa, sem_sb):
    cid = lax.axis_index("c")
    sid = lax.axis_index("s")
    wid = cid * 16 + sid
    bw = wid // (P // CHUNK)
    poff = (wid % (P // CHUNK)) * CHUNK
    pltpu.sync_copy(idx_hbm.at[bw, :, pl.ds(poff, CHUNK)], idx_v)
    pltpu.sync_copy(w_hbm.at[bw, :, pl.ds(poff, CHUNK)], w_v)
    pltpu.sync_copy(hull_hbm.at[bw, :, pl.ds(poff, CHUNK)], hull_v)
    pltpu.sync_copy(hmask_hbm, hm_v)
    mx = hm_v[0, :]
    my = hm_v[1, :]
    bufs = {0: (ra0, ra1, ra2, ra3, oa, sem_sa),
            1: (rb0, rb1, rb2, rb3, ob, sem_sb)}

    def fire(scn, side):
        for k in range(4):
            pltpu.async_copy(
                conv_hbm.at[idx_v.at[k, pl.ds(scn * SUB, SUB)]],
                bufs[side][k], sem)

    def wait_gathers(scn, side):
        for k in range(4):
            pltpu.make_async_copy(
                conv_hbm.at[idx_v.at[k, pl.ds(scn * SUB, SUB)]],
                bufs[side][k], sem).wait()

    def compute(scn, side):
        r0, r1, r2, r3, ov, _ = bufs[side]
        s0 = scn * SUB

        def bcast(vec, j):
            return plsc.bitcast(jnp.full((16,), vec[j]), jnp.bfloat16)

        def grp_body(g, carry2):
            gbase = g * 16
            wv0 = w_v[0, pl.ds(s0 + gbase, 16)]
            wv1 = w_v[1, pl.ds(s0 + gbase, 16)]
            wv2 = w_v[2, pl.ds(s0 + gbase, 16)]
            wv3 = w_v[3, pl.ds(s0 + gbase, 16)]
            hxv = hull_v[0, pl.ds(s0 + gbase, 16)]
            hyv = hull_v[1, pl.ds(s0 + gbase, 16)]
            for j in range(16):
                p = gbase + j
                wb0 = bcast(wv0, j)
                wb1 = bcast(wv1, j)
                wb2 = bcast(wv2, j)
                wb3 = bcast(wv3, j)
                for cc in range(TW // 32):
                    sl = pl.ds(cc * 32, 32)
                    acc = r0[p, sl] * wb0 + r1[p, sl] * wb1
                    acc = acc + r2[p, sl] * wb2 + r3[p, sl] * wb3
                    if cc == 4:
                        acc = acc + mx * bcast(hxv, j)
                        acc = acc + my * bcast(hyv, j)
                    ov[p, sl] = acc
            return carry2

        lax.fori_loop(0, SUB // 16, grp_body, 0)

    def store(scn, side):
        ov, sem_s = bufs[side][4], bufs[side][5]
        pltpu.async_copy(ov, out_hbm.at[pl.ds(wid * CHUNK + scn * SUB, SUB)],
                         sem_s)

    def wait_store(side):
        ov, sem_s = bufs[side][4], bufs[side][5]
        pltpu.make_async_copy(ov, out_hbm.at[pl.ds(wid * CHUNK, SUB)],
                              sem_s).wait()

    fire(0, 0)
    fire(1, 1)

    def pair_body(gg, carry):
        g0 = 2 * gg
        for side in (0, 1):
            g = g0 + side
            wait_gathers(g, side)

            @pl.when(gg >= 1)
            def _():
                wait_store(side)

            compute(g, side)

            @pl.when(g + 2 < NSUB)
            def _():
                fire(g + 2, side)

            store(g, side)
        return carry

    lax.fori_loop(0, NSUB // 2, pair_body, 0)
    wait_store(0)
    wait_store(1)


def _make_interp():
    mesh = plsc.VectorSubcoreMesh(core_axis_name="c", subcore_axis_name="s")
    rbuf = pltpu.VMEM((SUB, TW), jnp.bfloat16)
    return functools.partial(
        pl.kernel,
        out_type=jax.ShapeDtypeStruct((B * P, TW), jnp.bfloat16),
        mesh=mesh,
        scratch_types=[
            pltpu.VMEM((4, CHUNK), jnp.int32),
            pltpu.VMEM((4, CHUNK), jnp.int32),
            pltpu.VMEM((2, CHUNK), jnp.int32),
            pltpu.VMEM((2, 32), jnp.bfloat16),
            rbuf, rbuf, rbuf, rbuf, rbuf, rbuf, rbuf, rbuf,
            rbuf, rbuf,
            pltpu.SemaphoreType.DMA,
            pltpu.SemaphoreType.DMA,
            pltpu.SemaphoreType.DMA,
        ],
        compiler_params=pltpu.CompilerParams(use_tc_tiling_on_sc=False,
                                             needs_layout_passes=False),
    )(_interp_body)


def _make_prep():
    return pl.pallas_call(
        _prep_body,
        grid=(B,),
        in_specs=[
            pl.BlockSpec((1, P, 2), lambda b: (b, 0, 0)),
            pl.BlockSpec(memory_space=pltpu.SMEM),
        ],
        out_specs=[
            pl.BlockSpec((1, 4, P), lambda b: (b, 0, 0)),
            pl.BlockSpec((1, 4, P), lambda b: (b, 0, 0)),
        ],
        out_shape=[
            jax.ShapeDtypeStruct((B, 4, P), jnp.int32),
            jax.ShapeDtypeStruct((B, 4, P), jnp.int32),
        ],
    )


def _make_gcn():
    full = lambda b: (0, 0)
    return pl.pallas_call(
        _gcn_body,
        grid=(B,),
        in_specs=[
            pl.BlockSpec((1, P, TW), lambda b: (b, 0, 0)),
            pl.BlockSpec((1, 2, P), lambda b: (b, 0, 0)),
            pl.BlockSpec(memory_space=pltpu.SMEM),
            pl.BlockSpec((TW, 2 * HID), full),
            pl.BlockSpec((1, HID), full),
            pl.BlockSpec((HID, 2 * HID), full),
            pl.BlockSpec((1, HID), full),
            pl.BlockSpec((HID, 2), full),
            pl.BlockSpec(memory_space=pltpu.SMEM),
        ],
        out_specs=[
            pl.BlockSpec((1, 2, P), lambda b: (b, 0, 0)),
            pl.BlockSpec((1, 4, P), lambda b: (b, 0, 0)),
            pl.BlockSpec((1, 4, P), lambda b: (b, 0, 0)),
            pl.BlockSpec((1, 2, P), lambda b: (b, 0, 0)),
        ],
        out_shape=[
            jax.ShapeDtypeStruct((B, 2, P), jnp.float32),
            jax.ShapeDtypeStruct((B, 4, P), jnp.int32),
            jax.ShapeDtypeStruct((B, 4, P), jnp.int32),
            jax.ShapeDtypeStruct((B, 2, P), jnp.int32),
        ],
    )


def kernel(tg2, feature_hull, original_hull, binary_hull, bbox, dp,
           Ws0, Wn0, b0, Ws1, Wn1, b1, Wo, bo):
    del feature_hull, dp
    conv = jnp.pad(tg2.astype(jnp.bfloat16), ((0, 0), (0, TW - C), (0, 0), (0, 0)))
    conv = jnp.transpose(conv, (0, 2, 3, 1)).reshape(B * H * W, TW)

    w0cat = jnp.concatenate([Ws0, Wn0], axis=2)
    w0full = jnp.pad(w0cat, ((0, 0), (0, TW - FDIM), (0, 0))).astype(jnp.bfloat16)
    w1cat = jnp.concatenate([Ws1, Wn1], axis=2)

    prep = _make_prep()
    interp = _make_interp()
    gcn = _make_gcn()

    hmask = jnp.zeros((2, 32), jnp.float32)
    hmask = hmask.at[0, C - 128].set(1.0).at[1, C + 1 - 128].set(1.0)
    hmask = hmask.astype(jnp.bfloat16)

    idx4, wd = prep(original_hull, bbox)
    hull_t = jnp.transpose(binary_hull, (0, 2, 1))
    hld = _dup16(hull_t)
    preds = []
    for i in range(STEPS):
        cnn = interp(conv, idx4, wd, hld, hmask).reshape(B, P, TW)
        hull_t, idx4, wd, hld = gcn(cnn, hull_t, bbox,
                                    w0full[i], b0[i][None, :],
                                    w1cat[i], b1[i][None, :],
                                    Wo[i], bo[i][None, :])
        preds.append(hull_t)
    return jnp.transpose(jnp.stack(preds), (0, 1, 3, 2))


# speedup vs baseline: 1.1389x; 1.0394x over previous
import functools

import jax
import jax.numpy as jnp
from jax import lax
from jax.experimental import pallas as pl
from jax.experimental.pallas import tpu as pltpu
from jax.experimental.pallas import tpu_sc as plsc

B, P, H, W, C = 4, 8192, 128, 128, 130
STEPS, FDIM, HID = 3, 132, 64
TW = 160
NW = 32
CHUNK = (B * P) // NW
SUB = 128
NSUB = CHUNK // SUB


def _bilinear_prep(x, y, hs, ws, base):
    xs = x / hs * H
    ys = y / ws * W
    x0 = jnp.floor(xs)
    x1 = x0 + 1.0
    y0 = jnp.floor(ys)
    y1 = y0 + 1.0
    w00 = (x1 - xs) * (y1 - ys)
    w01 = (x1 - xs) * (ys - y0)
    w10 = (xs - x0) * (y1 - ys)
    w11 = (xs - x0) * (ys - y0)
    x0c = jnp.clip(x0, 0.0, H - 1.0)
    x1c = jnp.clip(x1, 0.0, H - 1.0)
    y0c = jnp.clip(y0, 0.0, W - 1.0)
    y1c = jnp.clip(y1, 0.0, W - 1.0)

    def flat(xi, yi):
        return (xi * W + yi).astype(jnp.int32) + base

    idx4 = jnp.stack([flat(x0c, y0c), flat(x0c, y1c),
                      flat(x1c, y0c), flat(x1c, y1c)], axis=0)
    w4 = jnp.stack([w00, w01, w10, w11], axis=0)
    return idx4, w4


def _dup16(x):
    u = jax.lax.bitcast_convert_type(
        x.astype(jnp.bfloat16), jnp.uint16).astype(jnp.uint32)
    return jax.lax.bitcast_convert_type(u * jnp.uint32(65537), jnp.int32)


def _prep_body(coords_ref, bbox_ref, idx_ref, w_ref):
    b = pl.program_id(0)
    x = coords_ref[0, :, 0]
    y = coords_ref[0, :, 1]
    idx4, w4 = _bilinear_prep(x, y, bbox_ref[b, 3], bbox_ref[b, 2],
                              b * (H * W))
    idx_ref[0] = idx4
    w_ref[0] = _dup16(w4)


def _ring_mean(y):
    n = y.shape[0]
    yy = jnp.concatenate([y[n - 4:], y, y[:4]], axis=0)
    s = None
    for j in range(9):
        if j == 4:
            continue
        part = yy[j:j + n]
        s = part if s is None else s + part
    return s * 0.125


def _gcn_body(cnn_ref, hull_ref, bbox_ref, w0_ref, b0_ref,
              w1_ref, b1_ref, wo_ref, bo_ref,
              pred_ref, idx_ref, w_ref, hld_ref):
    b = pl.program_id(0)
    x = cnn_ref[0]
    z = jnp.dot(x, w0_ref[...], preferred_element_type=jnp.float32)
    a = z[:, :HID] + b0_ref[0:1, :]
    h1 = jax.nn.relu(a + _ring_mean(z[:, HID:]))
    z2 = jnp.dot(h1, w1_ref[...], preferred_element_type=jnp.float32)
    a2 = z2[:, :HID] + b1_ref[0:1, :]
    h2 = jax.nn.relu(a2 + _ring_mean(z2[:, HID:]))
    shift = jnp.dot(h2, wo_ref[...], preferred_element_type=jnp.float32)
    pred_x = hull_ref[0, 0, :] + shift[:, 0] + bo_ref[0, 0]
    pred_y = hull_ref[0, 1, :] + shift[:, 1] + bo_ref[0, 1]
    pred_ref[0, 0, :] = pred_x
    pred_ref[0, 1, :] = pred_y
    hld_ref[0] = _dup16(jnp.stack([pred_x, pred_y], axis=0))

    hs = bbox_ref[b, 3]
    ws = bbox_ref[b, 2]
    cx = pred_x * hs
    cy = pred_y * ws
    idx4, w4 = _bilinear_prep(cx, cy, hs, ws, b * (H * W))
    idx_ref[0] = idx4
    w_ref[0] = _dup16(w4)


def _interp_body(conv_hbm, idx_hbm, w_hbm, hull_hbm, hmask_hbm, out_hbm,
                 idx_v, w_v, hull_v, hm_v,
                 ra0, ra1, ra2, ra3, rb0, rb1, rb2, rb3, oa, ob,
                 sem, sem_sa, sem_sb):
    cid = lax.axis_index("c")
    sid = lax.axis_index("s")
    wid = cid * 16 + sid
    bw = wid // (P // CHUNK)
    poff = (wid % (P // CHUNK)) * CHUNK
    pltpu.sync_copy(idx_hbm.at[bw, :, pl.ds(poff, CHUNK)], idx_v)
    pltpu.sync_copy(w_hbm.at[bw, :, pl.ds(poff, CHUNK)], w_v)
    pltpu.sync_copy(hull_hbm.at[bw, :, pl.ds(poff, CHUNK)], hull_v)
    pltpu.sync_copy(hmask_hbm, hm_v)
    mx = hm_v[0, :]
    my = hm_v[1, :]
    bufs = {0: (ra0, ra1, ra2, ra3, oa, sem_sa),
            1: (rb0, rb1, rb2, rb3, ob, sem_sb)}

    def fire(scn, side):
        for k in range(4):
            pltpu.async_copy(
                conv_hbm.at[idx_v.at[k, pl.ds(scn * SUB, SUB)]],
                bufs[side][k], sem)

    def wait_gathers(scn, side):
        for k in range(4):
            pltpu.make_async_copy(
                conv_hbm.at[idx_v.at[k, pl.ds(scn * SUB, SUB)]],
                bufs[side][k], sem).wait()

    def compute(scn, side):
        r0, r1, r2, r3, ov, _ = bufs[side]
        s0 = scn * SUB

        def bcast(vec, j):
            return plsc.bitcast(jnp.full((16,), vec[j]), jnp.bfloat16)

        def grp_body(g, carry2):
            gbase = g * 16
            wv0 = w_v[0, pl.ds(s0 + gbase, 16)]
            wv1 = w_v[1, pl.ds(s0 + gbase, 16)]
            wv2 = w_v[2, pl.ds(s0 + gbase, 16)]
            wv3 = w_v[3, pl.ds(s0 + gbase, 16)]
            hxv = hull_v[0, pl.ds(s0 + gbase, 16)]
            hyv = hull_v[1, pl.ds(s0 + gbase, 16)]
            for j in range(16):
                p = gbase + j
                wb0 = bcast(wv0, j)
                wb1 = bcast(wv1, j)
                wb2 = bcast(wv2, j)
                wb3 = bcast(wv3, j)
                for cc in range(TW // 32):
                    sl = pl.ds(cc * 32, 32)
                    acc = r0[p, sl] * wb0 + r1[p, sl] * wb1
                    acc = acc + r2[p, sl] * wb2 + r3[p, sl] * wb3
                    if cc == 4:
                        acc = acc + mx * bcast(hxv, j)
                        acc = acc + my * bcast(hyv, j)
                    ov[p, sl] = acc
            return carry2

        lax.fori_loop(0, SUB // 16, grp_body, 0)

    def store(scn, side):
        ov, sem_s = bufs[side][4], bufs[side][5]
        pltpu.async_copy(ov, out_hbm.at[pl.ds(wid * CHUNK + scn * SUB, SUB)],
                         sem_s)

    def wait_store(side):
        ov, sem_s = bufs[side][4], bufs[side][5]
        pltpu.make_async_copy(ov, out_hbm.at[pl.ds(wid * CHUNK, SUB)],
                              sem_s).wait()

    fire(0, 0)
    fire(1, 1)

    def pair_body(gg, carry):
        g0 = 2 * gg
        for side in (0, 1):
            g = g0 + side
            wait_gathers(g, side)

            @pl.when(gg >= 1)
            def _():
                wait_store(side)

            compute(g, side)

            @pl.when(g + 2 < NSUB)
            def _():
                fire(g + 2, side)

            store(g, side)
        return carry

    lax.fori_loop(0, NSUB // 2, pair_body, 0)
    wait_store(0)
    wait_store(1)


def _make_interp():
    mesh = plsc.VectorSubcoreMesh(core_axis_name="c", subcore_axis_name="s")
    rbuf = pltpu.VMEM((SUB, TW), jnp.bfloat16)
    return functools.partial(
        pl.kernel,
        out_type=jax.ShapeDtypeStruct((B * P, TW), jnp.bfloat16),
        mesh=mesh,
        scratch_types=[
            pltpu.VMEM((4, CHUNK), jnp.int32),
            pltpu.VMEM((4, CHUNK), jnp.int32),
            pltpu.VMEM((2, CHUNK), jnp.int32),
            pltpu.VMEM((2, 32), jnp.bfloat16),
            rbuf, rbuf, rbuf, rbuf, rbuf, rbuf, rbuf, rbuf,
            rbuf, rbuf,
            pltpu.SemaphoreType.DMA,
            pltpu.SemaphoreType.DMA,
            pltpu.SemaphoreType.DMA,
        ],
        compiler_params=pltpu.CompilerParams(use_tc_tiling_on_sc=False,
                                             needs_layout_passes=False),
    )(_interp_body)


def _make_prep():
    return pl.pallas_call(
        _prep_body,
        grid=(B,),
        in_specs=[
            pl.BlockSpec((1, P, 2), lambda b: (b, 0, 0)),
            pl.BlockSpec(memory_space=pltpu.SMEM),
        ],
        out_specs=[
            pl.BlockSpec((1, 4, P), lambda b: (b, 0, 0)),
            pl.BlockSpec((1, 4, P), lambda b: (b, 0, 0)),
        ],
        out_shape=[
            jax.ShapeDtypeStruct((B, 4, P), jnp.int32),
            jax.ShapeDtypeStruct((B, 4, P), jnp.int32),
        ],
    )


def _make_gcn():
    full = lambda b: (0, 0)
    return pl.pallas_call(
        _gcn_body,
        grid=(B,),
        in_specs=[
            pl.BlockSpec((1, P, TW), lambda b: (b, 0, 0)),
            pl.BlockSpec((1, 2, P), lambda b: (b, 0, 0)),
            pl.BlockSpec(memory_space=pltpu.SMEM),
            pl.BlockSpec((TW, 2 * HID), full),
            pl.BlockSpec((1, HID), full),
            pl.BlockSpec((HID, 2 * HID), full),
            pl.BlockSpec((1, HID), full),
            pl.BlockSpec((HID, 2), full),
            pl.BlockSpec(memory_space=pltpu.SMEM),
        ],
        out_specs=[
            pl.BlockSpec((1, 2, P), lambda b: (b, 0, 0)),
            pl.BlockSpec((1, 4, P), lambda b: (b, 0, 0)),
            pl.BlockSpec((1, 4, P), lambda b: (b, 0, 0)),
            pl.BlockSpec((1, 2, P), lambda b: (b, 0, 0)),
        ],
        out_shape=[
            jax.ShapeDtypeStruct((B, 2, P), jnp.float32),
            jax.ShapeDtypeStruct((B, 4, P), jnp.int32),
            jax.ShapeDtypeStruct((B, 4, P), jnp.int32),
            jax.ShapeDtypeStruct((B, 2, P), jnp.int32),
        ],
    )


def kernel(tg2, feature_hull, original_hull, binary_hull, bbox, dp,
           Ws0, Wn0, b0, Ws1, Wn1, b1, Wo, bo):
    del feature_hull, dp
    conv = jnp.pad(tg2.astype(jnp.bfloat16), ((0, 0), (0, TW - C), (0, 0), (0, 0)))
    conv = jnp.transpose(conv, (0, 2, 3, 1)).reshape(B * H * W, TW)

    w0cat = jnp.concatenate([Ws0, Wn0], axis=2)
    w0full = jnp.pad(w0cat, ((0, 0), (0, TW - FDIM), (0, 0))).astype(jnp.bfloat16)
    w1cat = jnp.concatenate([Ws1, Wn1], axis=2)

    prep = _make_prep()
    interp = _make_interp()
    gcn = _make_gcn()

    hmask = jnp.zeros((2, 32), jnp.float32)
    hmask = hmask.at[0, C - 128].set(1.0).at[1, C + 1 - 128].set(1.0)
    hmask = hmask.astype(jnp.bfloat16)

    idx4, wd = prep(original_hull, bbox)
    hull_t = jnp.transpose(binary_hull, (0, 2, 1))
    hld = _dup16(hull_t)
    preds = []
    for i in range(STEPS):
        cnn = interp(conv, idx4, wd, hld, hmask).reshape(B, P, TW)
        hull_t, idx4, wd, hld = gcn(cnn, hull_t, bbox,
                                    w0full[i], b0[i][None, :],
                                    w1cat[i], b1[i][None, :],
                                    Wo[i], bo[i][None, :])
        preds.append(hull_t)
    return jnp.transpose(jnp.stack(preds), (0, 1, 3, 2))
